# baseline (speedup 1.0000x reference)
import functools

import jax
import jax.numpy as jnp
from jax import lax
from jax.experimental import pallas as pl
from jax.experimental.pallas import tpu as pltpu
from jax.experimental.pallas import tpu_sc as plsc

NC = 2
NS = 16
NW = NC * NS
CHUNK = 128
NBUF = 4


@functools.lru_cache(maxsize=None)
def _build(n_rows: int, dim: int):
    rows_per_w = n_rows // NW
    n_chunks = rows_per_w // CHUNK
    n_rounds = n_chunks // NBUF
    assert n_rows % (NW * CHUNK * NBUF * 2) == 0
    total_chunk_rows = n_rows // CHUNK

    mesh = plsc.VectorSubcoreMesh(core_axis_name="c", subcore_axis_name="s")

    scratch = [pltpu.VMEM((n_chunks, CHUNK), jnp.int32)]
    scratch += [pltpu.VMEM((CHUNK, dim), jnp.float32) for _ in range(2 * NBUF)]
    scratch += [pltpu.SemaphoreType.DMA for _ in range(2 * NBUF)]
    scratch += [pltpu.SemaphoreType.DMA for _ in range(2 * NBUF)]

    @functools.partial(
        pl.kernel,
        out_type=jax.ShapeDtypeStruct((n_rows, dim), jnp.float32),
        mesh=mesh,
        scratch_types=scratch,
        compiler_params=pltpu.CompilerParams(use_tc_tiling_on_sc=False),
    )
    def emb(idx_hbm, table_hbm, out_hbm, idx_v, *rest):
        bufs = [[rest[2 * b + p] for p in range(2)] for b in range(NBUF)]
        o = 2 * NBUF
        gsem = [[rest[o + 2 * b + p] for p in range(2)] for b in range(NBUF)]
        o = 4 * NBUF
        psem = [[rest[o + 2 * b + p] for p in range(2)] for b in range(NBUF)]

        wid = lax.axis_index("s") * NC + lax.axis_index("c")
        base = wid * rows_per_w
        chunk0 = wid * n_chunks

        pltpu.sync_copy(idx_hbm.at[pl.ds(chunk0, n_chunks)], idx_v)

        def gather(g, b, p):
            return pltpu.async_copy(
                table_hbm.at[idx_v.at[g]], bufs[b][p], gsem[b][p])

        def gather_wait(g, b, p):
            pltpu.make_async_copy(
                table_hbm.at[idx_v.at[g]], bufs[b][p], gsem[b][p]).wait()

        def put(g, b, p):
            return pltpu.async_copy(
                bufs[b][p], out_hbm.at[pl.ds(base + g * CHUNK, CHUNK)],
                psem[b][p])

        def put_wait(g, b, p):
            pltpu.make_async_copy(
                bufs[b][p], out_hbm.at[pl.ds(base + g * CHUNK, CHUNK)],
                psem[b][p]).wait()

        for b in range(NBUF):
            gather(b, b, 0)
        for b in range(NBUF):
            gather_wait(b, b, 0)
            put(b, b, 0)
            gather(NBUF + b, b, 1)

        def round_(r, p):
            for b in range(NBUF):
                g = r * NBUF + b
                gather_wait(g, b, p)
                put(g, b, p)
                put_wait(g - NBUF, b, 1 - p)
                gather(g + NBUF, b, 1 - p)

        def body(k, carry):
            round_(2 * k + 1, 1)
            round_(2 * k + 2, 0)
            return carry

        lax.fori_loop(0, (n_rounds - 2) // 2, body, 0)

        r = n_rounds - 1
        for b in range(NBUF):
            g = r * NBUF + b
            gather_wait(g, b, 1)
            put(g, b, 1)
            put_wait(g - NBUF, b, 0)
        for b in range(NBUF):
            put_wait(r * NBUF + b, b, 1)

    def run(x, weight):
        idx = x.reshape(total_chunk_rows, CHUNK).astype(jnp.int32)
        out = emb(idx, weight)
        return out

    return run


def kernel(x, weight):
    b, h = x.shape
    v, d = weight.shape
    run = _build(b * h, d)
    return run(x, weight).reshape(b, h, d)
